# Initial kernel scaffold; baseline (speedup 1.0000x reference)
import jax
import jax.numpy as jnp
from jax.experimental import pallas as pl


def kernel(user_emb, item_emb, W_ih, W_hh, b_ih, b_hh, W1, fc0_W, fc0_b, fc1_W, fc1_b, W2, uids, padded_seqs, lens, cur_sidx, g0_src, g0_dst, g1_src, g1_dst, idx_map0, idx_map1):
    raise NotImplementedError("write your pallas kernel here")


# speedup vs baseline: 2.0792x; 2.0792x over previous
import functools

import jax
import jax.numpy as jnp
from jax import lax
from jax.experimental import pallas as pl
from jax.experimental.pallas import tpu as pltpu

N0, N1, N2 = 10000, 4096, 512
E0, E1 = 131072, 16384
D, L = 128, 20
NUM_ITEMS = 100000

B0 = 400
G0 = N0 // B0
WI = 4096
GI = 25
WT = NUM_ITEMS - 24 * WI


def _lstm_feat_body(x_ref, lt_ref, lens_ref, cs_ref, Wih_ref, Whh_ref, b_ref,
                    W1_ref, st_ref, feat_ref, mx_ref):
    x = x_ref[...]
    n = jnp.sqrt(jnp.sum(x * x, axis=-1, keepdims=True))
    x = x * (1.0 / jnp.maximum(n, 1.0))

    Wih = Wih_ref[...]
    Whh = Whh_ref[...]
    b = b_ref[...]
    dn = (((1,), (1,)), ((), ()))

    lens = lens_ref[0]
    h = jnp.zeros((B0, D), jnp.float32)
    c = jnp.zeros((B0, D), jnp.float32)
    hl = jnp.zeros((B0, D), jnp.float32)
    for t in range(L):
        xt = x[:, t, :]
        gates = (lax.dot_general(xt, Wih, dn, preferred_element_type=jnp.float32)
                 + lax.dot_general(h, Whh, dn, preferred_element_type=jnp.float32)
                 + b[None, :])
        i = jax.nn.sigmoid(gates[:, :D])
        f = jax.nn.sigmoid(gates[:, D:2 * D])
        g = jnp.tanh(gates[:, 2 * D:3 * D])
        o = jax.nn.sigmoid(gates[:, 3 * D:])
        c = f * c + i * g
        h = o * jnp.tanh(c)
        hl = jnp.where(lens == t + 1, h, hl)

    st_ref[...] = hl

    lt = lt_ref[...]
    ln = jnp.sqrt(jnp.sum(lt * lt, axis=-1, keepdims=True))
    lt = lt * (1.0 / jnp.maximum(ln, 1.0))

    W1 = W1_ref[...]
    feat = (lax.dot_general(lt, W1[:, :D], dn, preferred_element_type=jnp.float32)
            + lax.dot_general(hl, W1[:, D:], dn, preferred_element_type=jnp.float32))
    feat = jnp.maximum(feat, 0.0)

    rows = pl.program_id(0) * B0 + lax.broadcasted_iota(jnp.int32, (B0, 1), 0)
    cs = cs_ref[...]
    is_cur = jnp.any(rows == cs[None, :], axis=1, keepdims=True)
    feat = jnp.where(is_cur, hl, feat)
    feat_ref[...] = feat

    nf2 = jnp.max(jnp.sum(feat * feat, axis=-1))
    prev = jnp.where(pl.program_id(0) == 0, 0.0, mx_ref[0])
    mx_ref[0] = jnp.maximum(prev, nf2)


def _lstm_feat(emb_rows, lt_rows, lens, cur_sidx, W_ih, W_hh, b_sum, W1):
    full = lambda s: pl.BlockSpec(s, lambda i: (0,) * len(s))
    return pl.pallas_call(
        _lstm_feat_body,
        grid=(G0,),
        in_specs=[
            pl.BlockSpec((B0, L, D), lambda i: (i, 0, 0)),
            pl.BlockSpec((B0, D), lambda i: (i, 0)),
            pl.BlockSpec((1, B0, 1), lambda i: (i, 0, 0)),
            full((N2,)),
            full((4 * D, D)),
            full((4 * D, D)),
            full((4 * D,)),
            full((D, 2 * D)),
        ],
        out_specs=[
            pl.BlockSpec((B0, D), lambda i: (i, 0)),
            pl.BlockSpec((B0, D), lambda i: (i, 0)),
            pl.BlockSpec(memory_space=pltpu.SMEM, block_shape=(1,),
                         index_map=lambda i: (0,)),
        ],
        out_shape=[
            jax.ShapeDtypeStruct((N0, D), jnp.float32),
            jax.ShapeDtypeStruct((N0, D), jnp.float32),
            jax.ShapeDtypeStruct((1,), jnp.float32),
        ],
    )(emb_rows, lt_rows, lens.reshape(G0, B0, 1), cur_sidx, W_ih, W_hh, b_sum, W1)


def _logits_body(sr_ref, items_ref, out_ref, buf, sems, obuf, osems, tbuf, tsem):
    gi = pl.program_id(0)
    slot = gi % 2
    last = GI - 1

    def in_full(s, step):
        return pltpu.make_async_copy(
            items_ref.at[pl.ds(step * WI, WI), :], buf.at[s], sems.at[s])

    def in_tail(s):
        return pltpu.make_async_copy(
            items_ref.at[pl.ds(last * WI, WT), :],
            buf.at[s, pl.ds(0, WT), :], sems.at[s])

    def out_copy(s, step):
        return pltpu.make_async_copy(
            obuf.at[s], out_ref.at[:, pl.ds(step * WI, WI)], osems.at[s])

    @pl.when(gi == 0)
    def _():
        in_full(0, 0).start()

    @pl.when(gi + 1 < last)
    def _():
        in_full((gi + 1) % 2, gi + 1).start()

    @pl.when(gi + 1 == last)
    def _():
        in_tail((gi + 1) % 2).start()

    @pl.when(gi < last)
    def _():
        in_full(slot, gi).wait()

    @pl.when(gi == last)
    def _():
        in_tail(slot).wait()
    sr = sr_ref[...].astype(jnp.bfloat16)

    def rnorm(x):
        n = jnp.sqrt(jnp.sum(x * x, axis=-1, keepdims=True))
        return (x * (1.0 / jnp.maximum(n, 1.0))).astype(jnp.bfloat16)

    dn = (((1,), (1,)), ((), ()))

    @pl.when(gi < last)
    def _():
        @pl.when(gi >= 2)
        def _():
            out_copy(slot, gi - 2).wait()
        obuf[slot] = lax.dot_general(sr, rnorm(buf[slot]), dn,
                                     preferred_element_type=jnp.float32)
        out_copy(slot, gi).start()

    @pl.when(gi == last)
    def _():
        tbuf[...] = lax.dot_general(sr, rnorm(buf[slot, :WT, :]), dn,
                                    preferred_element_type=jnp.float32)
        pltpu.make_async_copy(
            tbuf, out_ref.at[:, pl.ds(last * WI, WT)], tsem).start()
        out_copy((gi + 1) % 2, gi - 1).wait()
        out_copy(slot, gi - 2).wait()
        pltpu.make_async_copy(
            tbuf, out_ref.at[:, pl.ds(last * WI, WT)], tsem).wait()


def _logits(sr, items1):
    return pl.pallas_call(
        _logits_body,
        grid=(GI,),
        in_specs=[
            pl.BlockSpec((N2, D), lambda i: (0, 0)),
            pl.BlockSpec(memory_space=pl.ANY),
        ],
        out_specs=pl.BlockSpec(memory_space=pl.ANY),
        out_shape=jax.ShapeDtypeStruct((N2, NUM_ITEMS), jnp.float32),
        scratch_shapes=[
            pltpu.VMEM((2, WI, D), jnp.float32),
            pltpu.SemaphoreType.DMA((2,)),
            pltpu.VMEM((2, N2, WI), jnp.float32),
            pltpu.SemaphoreType.DMA((2,)),
            pltpu.VMEM((N2, WT), jnp.float32),
            pltpu.SemaphoreType.DMA,
        ],
    )(sr, items1)


def _gat(feat, idx_map, src, dst, fc_W, fc_b, n_dst, bound):
    feat_dst = feat[idx_map]
    s = jnp.sum(feat[src] * feat_dst[dst], axis=-1)
    w = jnp.exp(s - bound)
    den = jax.ops.segment_sum(w, dst, num_segments=n_dst)
    acc = jax.ops.segment_sum(feat[src] * w[:, None], dst, num_segments=n_dst)
    rst = acc / jnp.maximum(den, 1e-38)[:, None]
    rst = jax.nn.relu(rst @ fc_W.T + fc_b)
    return feat_dst + rst


def kernel(user_emb, item_emb, W_ih, W_hh, b_ih, b_hh, W1, fc0_W, fc0_b,
           fc1_W, fc1_b, W2, uids, padded_seqs, lens, cur_sidx,
           g0_src, g0_dst, g1_src, g1_dst, idx_map0, idx_map1):
    emb_rows = item_emb[padded_seqs]
    lt_rows = user_emb[uids]
    short_term, feat, mx = _lstm_feat(
        emb_rows, lt_rows, lens, cur_sidx, W_ih, W_hh, b_ih + b_hh, W1)
    bound0 = mx[0]

    feat1 = _gat(feat, idx_map0, g0_src, g0_dst, fc0_W, fc0_b, N1, bound0)
    bound1 = jnp.max(jnp.sum(feat1 * feat1, axis=-1))
    feat2 = _gat(feat1, idx_map1, g1_src, g1_dst, fc1_W, fc1_b, N2, bound1)

    cus = short_term[cur_sidx]
    sr = cus @ W2[:, :D].T + feat2 @ W2[:, D:].T
    return _logits(sr, item_emb[1:])


# speedup vs baseline: 5.1880x; 2.4951x over previous
import functools

import jax
import jax.numpy as jnp
from jax import lax
from jax.experimental import pallas as pl
from jax.experimental.pallas import tpu as pltpu
from jax.experimental.pallas import tpu_sc as plsc

N0, N1, N2 = 10000, 4096, 512
E0, E1 = 131072, 16384
D, L = 128, 20
NUM_ITEMS = 100000

B0 = 400
G0 = N0 // B0
WI = 4096
GI = 25
WT = NUM_ITEMS - 24 * WI


def _lstm_feat_body(x_ref, lt_ref, lens_ref, cs_ref, Wih_ref, Whh_ref, b_ref,
                    W1_ref, st_ref, feat_ref, mx_ref):
    x = x_ref[...]
    n = jnp.sqrt(jnp.sum(x * x, axis=-1, keepdims=True))
    x = x * (1.0 / jnp.maximum(n, 1.0))

    Wih = Wih_ref[...]
    Whh = Whh_ref[...]
    b = b_ref[...]
    dn = (((1,), (1,)), ((), ()))

    lens = lens_ref[0]
    h = jnp.zeros((B0, D), jnp.float32)
    c = jnp.zeros((B0, D), jnp.float32)
    hl = jnp.zeros((B0, D), jnp.float32)
    for t in range(L):
        xt = x[:, t, :]
        gates = (lax.dot_general(xt, Wih, dn, preferred_element_type=jnp.float32)
                 + lax.dot_general(h, Whh, dn, preferred_element_type=jnp.float32)
                 + b[None, :])
        i = jax.nn.sigmoid(gates[:, :D])
        f = jax.nn.sigmoid(gates[:, D:2 * D])
        g = jnp.tanh(gates[:, 2 * D:3 * D])
        o = jax.nn.sigmoid(gates[:, 3 * D:])
        c = f * c + i * g
        h = o * jnp.tanh(c)
        hl = jnp.where(lens == t + 1, h, hl)

    st_ref[...] = hl

    lt = lt_ref[...]
    ln = jnp.sqrt(jnp.sum(lt * lt, axis=-1, keepdims=True))
    lt = lt * (1.0 / jnp.maximum(ln, 1.0))

    W1 = W1_ref[...]
    feat = (lax.dot_general(lt, W1[:, :D], dn, preferred_element_type=jnp.float32)
            + lax.dot_general(hl, W1[:, D:], dn, preferred_element_type=jnp.float32))
    feat = jnp.maximum(feat, 0.0)

    rows = pl.program_id(0) * B0 + lax.broadcasted_iota(jnp.int32, (B0, 1), 0)
    cs = cs_ref[...]
    is_cur = jnp.any(rows == cs[None, :], axis=1, keepdims=True)
    feat = jnp.where(is_cur, hl, feat)
    feat_ref[...] = feat

    nf2 = jnp.max(jnp.sum(feat * feat, axis=-1))
    prev = jnp.where(pl.program_id(0) == 0, 0.0, mx_ref[0])
    mx_ref[0] = jnp.maximum(prev, nf2)


def _lstm_feat(emb_rows, lt_rows, lens, cur_sidx, W_ih, W_hh, b_sum, W1):
    full = lambda s: pl.BlockSpec(s, lambda i: (0,) * len(s))
    return pl.pallas_call(
        _lstm_feat_body,
        grid=(G0,),
        in_specs=[
            pl.BlockSpec((B0, L, D), lambda i: (i, 0, 0)),
            pl.BlockSpec((B0, D), lambda i: (i, 0)),
            pl.BlockSpec((1, B0, 1), lambda i: (i, 0, 0)),
            full((N2,)),
            full((4 * D, D)),
            full((4 * D, D)),
            full((4 * D,)),
            full((D, 2 * D)),
        ],
        out_specs=[
            pl.BlockSpec((B0, D), lambda i: (i, 0)),
            pl.BlockSpec((B0, D), lambda i: (i, 0)),
            pl.BlockSpec(memory_space=pltpu.SMEM, block_shape=(1,),
                         index_map=lambda i: (0,)),
        ],
        out_shape=[
            jax.ShapeDtypeStruct((N0, D), jnp.float32),
            jax.ShapeDtypeStruct((N0, D), jnp.float32),
            jax.ShapeDtypeStruct((1,), jnp.float32),
        ],
    )(emb_rows, lt_rows, lens.reshape(G0, B0, 1), cur_sidx, W_ih, W_hh, b_sum, W1)


def _logits_body(sr_ref, items_ref, out_ref, buf, sems, obuf, osems, tbuf, tsem):
    gi = pl.program_id(0)
    slot = gi % 2
    last = GI - 1

    def in_full(s, step):
        return pltpu.make_async_copy(
            items_ref.at[pl.ds(step * WI, WI), :], buf.at[s], sems.at[s])

    def in_tail(s):
        return pltpu.make_async_copy(
            items_ref.at[pl.ds(last * WI, WT), :],
            buf.at[s, pl.ds(0, WT), :], sems.at[s])

    def out_copy(s, step):
        return pltpu.make_async_copy(
            obuf.at[s], out_ref.at[:, pl.ds(step * WI, WI)], osems.at[s])

    @pl.when(gi == 0)
    def _():
        in_full(0, 0).start()

    @pl.when(gi + 1 < last)
    def _():
        in_full((gi + 1) % 2, gi + 1).start()

    @pl.when(gi + 1 == last)
    def _():
        in_tail((gi + 1) % 2).start()

    @pl.when(gi < last)
    def _():
        in_full(slot, gi).wait()

    @pl.when(gi == last)
    def _():
        in_tail(slot).wait()
    sr = sr_ref[...].astype(jnp.bfloat16)

    def rnorm(x):
        n = jnp.sqrt(jnp.sum(x * x, axis=-1, keepdims=True))
        return (x * (1.0 / jnp.maximum(n, 1.0))).astype(jnp.bfloat16)

    dn = (((1,), (1,)), ((), ()))

    @pl.when(gi < last)
    def _():
        @pl.when(gi >= 2)
        def _():
            out_copy(slot, gi - 2).wait()
        obuf[slot] = lax.dot_general(sr, rnorm(buf[slot]), dn,
                                     preferred_element_type=jnp.float32)
        out_copy(slot, gi).start()

    @pl.when(gi == last)
    def _():
        tbuf[...] = lax.dot_general(sr, rnorm(buf[slot, :WT, :]), dn,
                                    preferred_element_type=jnp.float32)
        pltpu.make_async_copy(
            tbuf, out_ref.at[:, pl.ds(last * WI, WT)], tsem).start()
        out_copy((gi + 1) % 2, gi - 1).wait()
        out_copy(slot, gi - 2).wait()
        pltpu.make_async_copy(
            tbuf, out_ref.at[:, pl.ds(last * WI, WT)], tsem).wait()


def _logits(sr, items1):
    return pl.pallas_call(
        _logits_body,
        grid=(GI,),
        in_specs=[
            pl.BlockSpec((N2, D), lambda i: (0, 0)),
            pl.BlockSpec(memory_space=pl.ANY),
        ],
        out_specs=pl.BlockSpec(memory_space=pl.ANY),
        out_shape=jax.ShapeDtypeStruct((N2, NUM_ITEMS), jnp.float32),
        scratch_shapes=[
            pltpu.VMEM((2, WI, D), jnp.float32),
            pltpu.SemaphoreType.DMA((2,)),
            pltpu.VMEM((2, N2, WI), jnp.float32),
            pltpu.SemaphoreType.DMA((2,)),
            pltpu.VMEM((N2, WT), jnp.float32),
            pltpu.SemaphoreType.DMA,
        ],
    )(sr, items1)


NC, NS = 2, 16
NW = NC * NS
CH = 32


def _gat_sc(feat, fd, src, dst, bnd16, n_dst, n_edges):
    nch = n_edges // (NW * CH)
    stripe = n_dst // NS
    mesh = plsc.VectorSubcoreMesh(core_axis_name="c", subcore_axis_name="s")

    def body(feat_hbm, fd_hbm, src_hbm, dst_hbm, bnd_hbm, out_hbm, den_hbm,
             sidx, didx, rs, rd, sb, bndv, denf, dout, acc_sh,
             sem_s0, sem_s1, sem_d0, sem_d1, sem_a0, sem_a1):
        c = lax.axis_index("c")
        s = lax.axis_index("s")
        wid = c * NS + s
        sems = ((sem_s0, sem_d0, sem_a0), (sem_s1, sem_d1, sem_a1))
        z16 = jnp.zeros((16,), jnp.float32)
        lane = lax.iota(jnp.int32, 16)

        pltpu.sync_copy(src_hbm.at[wid], sidx)
        pltpu.sync_copy(dst_hbm.at[wid], didx)
        pltpu.sync_copy(bnd_hbm, bndv)
        bnd = bndv[...]

        def zden(r, _):
            for j in range(16):
                denf[pl.ds(r * 256 + j * 16, 16)] = z16
            return 0
        lax.fori_loop(0, (8 * n_dst) // 256, zden, 0)

        def zrow(r, _):
            for j in range(D // 16):
                sb[0, r, pl.ds(j * 16, 16)] = z16
            return 0
        lax.fori_loop(0, CH, zrow, 0)
        if stripe >= CH:
            for k in range(stripe // CH):
                pltpu.sync_copy(
                    sb.at[0], acc_sh.at[pl.ds(s * stripe + k * CH, CH)])
        else:
            pltpu.sync_copy(sb.at[0, pl.ds(0, stripe)],
                            acc_sh.at[pl.ds(s * stripe, stripe)])
        plsc.subcore_barrier()

        def g_src(ci, b):
            return pltpu.make_async_copy(
                feat_hbm.at[sidx.at[ci]], rs.at[b], sems[b][0])

        def g_dst(ci, b):
            return pltpu.make_async_copy(
                fd_hbm.at[didx.at[ci]], rd.at[b], sems[b][1])

        def g_acc(ci, b):
            return pltpu.make_async_copy(
                sb.at[b], acc_sh.at[didx.at[ci]], sems[b][2])

        for b in range(2):
            g_src(b, b).start()
            g_dst(b, b).start()

        def chunk(i, _):
            for b in range(2):
                ci = 2 * i + b
                g_src(ci, b).wait()
                g_dst(ci, b).wait()

                @pl.when(i >= 1)
                def _():
                    g_acc(2 * (i - 1) + b, b).wait()

                def group(g, _):
                    idx_e = lane + g * 16
                    sc = z16
                    def dot8(k, s):
                        for u in range(8):
                            dd = (lane + (k * 8 + u)) & (D - 1)
                            a = plsc.load_gather(rs.at[b], [idx_e, dd])
                            bb = plsc.load_gather(rd.at[b], [idx_e, dd])
                            s = s + a * bb
                        return s
                    sc = lax.fori_loop(0, D // 8, dot8, sc)
                    w = jnp.exp(sc - bnd)
                    dst16 = plsc.load_gather(didx.at[ci], [idx_e])
                    dslot = (lane & 7) * n_dst + dst16
                    plsc.addupdate_scatter(denf, [dslot], w, mask=lane < 8)
                    plsc.addupdate_scatter(denf, [dslot], w, mask=lane >= 8)

                    def scl8(k, _):
                        for u in range(8):
                            dd = (lane + (k * 8 + u)) & (D - 1)
                            v = plsc.load_gather(rs.at[b], [idx_e, dd]) * w
                            plsc.store_scatter(sb.at[b], [idx_e, dd], v)
                        return 0
                    lax.fori_loop(0, D // 8, scl8, 0)
                    return 0
                lax.fori_loop(0, CH // 16, group, 0)

                pltpu.async_copy(
                    sb.at[b], acc_sh.at[didx.at[ci]], sems[b][2], add=True)

                @pl.when(ci + 2 < nch)
                def _():
                    g_src(ci + 2, b).start()
                    g_dst(ci + 2, b).start()
            return 0
        lax.fori_loop(0, nch // 2, chunk, 0)

        for b in range(2):
            g_acc(nch - 2 + b, b).wait()

        def dred(j, _):
            acc = z16
            for l in range(8):
                acc = acc + denf[pl.ds(l * n_dst + j * 16, 16)]
            dout[pl.ds(j * 16, 16)] = acc
            return 0
        lax.fori_loop(0, n_dst // 16, dred, 0)
        pltpu.sync_copy(dout, den_hbm.at[wid])

        plsc.subcore_barrier()
        if stripe >= CH:
            for k in range(stripe // CH):
                pltpu.sync_copy(
                    acc_sh.at[pl.ds(s * stripe + k * CH, CH)],
                    out_hbm.at[c, pl.ds(s * stripe + k * CH, CH)])
        else:
            pltpu.sync_copy(acc_sh.at[pl.ds(s * stripe, stripe)],
                            out_hbm.at[c, pl.ds(s * stripe, stripe)])

    f = pl.kernel(
        body,
        out_type=[jax.ShapeDtypeStruct((NC, n_dst, D), jnp.float32),
                  jax.ShapeDtypeStruct((NW, n_dst), jnp.float32)],
        mesh=mesh,
        compiler_params=pltpu.CompilerParams(needs_layout_passes=False),
        scratch_types=[
            pltpu.VMEM((nch, CH), jnp.int32),
            pltpu.VMEM((nch, CH), jnp.int32),
            pltpu.VMEM((2, CH, D), jnp.float32),
            pltpu.VMEM((2, CH, D), jnp.float32),
            pltpu.VMEM((2, CH, D), jnp.float32),
            pltpu.VMEM((16,), jnp.float32),
            pltpu.VMEM((8 * n_dst,), jnp.float32),
            pltpu.VMEM((n_dst,), jnp.float32),
            pltpu.VMEM_SHARED((n_dst, D), jnp.float32),
            pltpu.SemaphoreType.DMA, pltpu.SemaphoreType.DMA,
            pltpu.SemaphoreType.DMA, pltpu.SemaphoreType.DMA,
            pltpu.SemaphoreType.DMA, pltpu.SemaphoreType.DMA,
        ],
    )
    return f(feat, fd, src.reshape(NW, nch, CH), dst.reshape(NW, nch, CH),
             bnd16)


def _gat_fin_body(acc_ref, den_ref, fd_ref, W_ref, b_ref, out_ref, mx_ref):
    a = acc_ref[0] + acc_ref[1]
    den = jnp.maximum(jnp.sum(den_ref[...], axis=1, keepdims=True), 1e-30)
    rst = a / den
    dn = (((1,), (1,)), ((), ()))
    z = lax.dot_general(rst, W_ref[...], dn,
                        preferred_element_type=jnp.float32) + b_ref[...][None, :]
    out = fd_ref[...] + jnp.maximum(z, 0.0)
    out_ref[...] = out
    nf2 = jnp.max(jnp.sum(out * out, axis=-1))
    prev = jnp.where(pl.program_id(0) == 0, 0.0, mx_ref[0])
    mx_ref[0] = jnp.maximum(prev, nf2)


def _gat_fin(acc, den, fd, fc_W, fc_b, n_dst):
    BD = 512
    full = lambda s: pl.BlockSpec(s, lambda i: (0,) * len(s))
    return pl.pallas_call(
        _gat_fin_body,
        grid=(n_dst // BD,),
        in_specs=[
            pl.BlockSpec((NC, BD, D), lambda i: (0, i, 0)),
            pl.BlockSpec((BD, NW), lambda i: (i, 0)),
            pl.BlockSpec((BD, D), lambda i: (i, 0)),
            full((D, D)),
            full((D,)),
        ],
        out_specs=[
            pl.BlockSpec((BD, D), lambda i: (i, 0)),
            pl.BlockSpec(memory_space=pltpu.SMEM, block_shape=(1,),
                         index_map=lambda i: (0,)),
        ],
        out_shape=[
            jax.ShapeDtypeStruct((n_dst, D), jnp.float32),
            jax.ShapeDtypeStruct((1,), jnp.float32),
        ],
    )(acc, den, fd, fc_W, fc_b)


def _gat(feat, idx_map, src, dst, fc_W, fc_b, n_dst, bound):
    feat_dst = feat[idx_map]
    s = jnp.sum(feat[src] * feat_dst[dst], axis=-1)
    w = jnp.exp(s - bound)
    den = jax.ops.segment_sum(w, dst, num_segments=n_dst)
    acc = jax.ops.segment_sum(feat[src] * w[:, None], dst, num_segments=n_dst)
    rst = acc / jnp.maximum(den, 1e-38)[:, None]
    rst = jax.nn.relu(rst @ fc_W.T + fc_b)
    return feat_dst + rst


def kernel(user_emb, item_emb, W_ih, W_hh, b_ih, b_hh, W1, fc0_W, fc0_b,
           fc1_W, fc1_b, W2, uids, padded_seqs, lens, cur_sidx,
           g0_src, g0_dst, g1_src, g1_dst, idx_map0, idx_map1):
    emb_rows = item_emb[padded_seqs]
    lt_rows = user_emb[uids]
    short_term, feat, mx = _lstm_feat(
        emb_rows, lt_rows, lens, cur_sidx, W_ih, W_hh, b_ih + b_hh, W1)
    fd0 = feat[idx_map0]
    acc0, den0 = _gat_sc(feat, fd0, g0_src, g0_dst,
                         jnp.full((16,), mx[0]), N1, E0)
    feat1, mx1 = _gat_fin(acc0, den0.T, fd0, fc0_W, fc0_b, N1)
    fd1 = feat1[idx_map1]
    acc1, den1 = _gat_sc(feat1, fd1, g1_src, g1_dst,
                         jnp.full((16,), mx1[0]), N2, E1)
    feat2, _ = _gat_fin(acc1, den1.T, fd1, fc1_W, fc1_b, N2)

    cus = short_term[cur_sidx]
    sr = cus @ W2[:, :D].T + feat2 @ W2[:, D:].T
    return _logits(sr, item_emb[1:])
